# baseline (speedup 1.0000x reference)
import functools

import jax
import jax.numpy as jnp
from jax import lax
from jax.experimental import pallas as pl
from jax.experimental.pallas import tpu as pltpu
from jax.experimental.pallas import tpu_sc as plsc

N_FIELDS = 26
VOCAB = 100000
EMB = 32
BATCH = 16384

_NW = 32
_BW = BATCH // _NW
_BB = 64
_NB = _BW // _BB
_OUTW = N_FIELDS * EMB
_XW = _BW * N_FIELDS


def _body(xf, tab, out, xv, idx2, rows2, asm, sem0, sem1):
    wid = lax.axis_index("s") * 2 + lax.axis_index("c")
    b0 = wid * _BW
    pltpu.sync_copy(xf.at[pl.ds(wid * _XW, _XW)], xv)

    lane = lax.iota(jnp.int32, 16)
    sems = (sem0, sem1)

    def fire(bb, f, buf):
        for s in range(_BB // 16):
            flat = (bb * _BB + s * 16 + lane) * N_FIELDS + f
            idx2[buf, pl.ds(s * 16, 16)] = plsc.load_gather(xv, [flat])
        return pltpu.async_copy(
            tab.at[f].at[idx2.at[buf]], rows2.at[buf], sems[buf]
        )

    def block(bb, _):
        fire(bb, 0, 0)
        for f in range(N_FIELDS):
            buf = f % 2
            if f + 1 < N_FIELDS:
                fire(bb, f + 1, 1 - buf)
            pltpu.make_async_copy(
                tab.at[f].at[idx2.at[buf]], rows2.at[buf], sems[buf]
            ).wait()

            def interleave(t, _):
                for k in range(8):
                    for c in (0, 16):
                        asm[t * 8 + k, pl.ds(f * EMB + c, 16)] = (
                            rows2[buf, t * 8 + k, pl.ds(c, 16)]
                        )
                return ()

            lax.fori_loop(0, _BB // 8, interleave, ())
        pltpu.sync_copy(asm, out.at[pl.ds(b0 + bb * _BB, _BB)])
        return ()

    lax.fori_loop(0, _NB, block, ())


@jax.jit
def kernel(x, tables):
    xf = x.reshape(BATCH * N_FIELDS)
    mesh = plsc.VectorSubcoreMesh(core_axis_name="c", subcore_axis_name="s")
    run = functools.partial(
        pl.kernel,
        mesh=mesh,
        compiler_params=pltpu.CompilerParams(
            use_tc_tiling_on_sc=False, needs_layout_passes=False
        ),
        out_type=jax.ShapeDtypeStruct((BATCH, _OUTW), jnp.float32),
        scratch_types=[
            pltpu.VMEM((_XW,), jnp.int32),
            pltpu.VMEM((2, _BB), jnp.int32),
            pltpu.VMEM((2, _BB, EMB), jnp.float32),
            pltpu.VMEM((_BB, _OUTW), jnp.float32),
            pltpu.SemaphoreType.DMA,
            pltpu.SemaphoreType.DMA,
        ],
    )(_body)
    return run(xf, tables)


# speedup vs baseline: 1.0177x; 1.0177x over previous
import functools

import jax
import jax.numpy as jnp
from jax import lax
from jax.experimental import pallas as pl
from jax.experimental.pallas import tpu as pltpu
from jax.experimental.pallas import tpu_sc as plsc

N_FIELDS = 26
VOCAB = 100000
EMB = 32
BATCH = 16384

_NW = 32
_ROWS_W = BATCH * N_FIELDS // _NW
_G = 128
_NG = _ROWS_W // _G


def _body(tab, xf, out, xv, idxg, rows, sem):
    wid = lax.axis_index("s") * 2 + lax.axis_index("c")
    base = wid * _ROWS_W
    pltpu.sync_copy(xf.at[pl.ds(base, _ROWS_W)], xv)

    lane = lax.iota(jnp.int32, 16)

    def step(g, _):
        for s in range(_G // 16):
            j0 = g * _G + s * 16
            pos = j0 + lane
            off = (pos % N_FIELDS) * VOCAB
            idxg[pl.ds(s * 16, 16)] = xv[pl.ds(j0, 16)] + off
        pltpu.async_copy(tab.at[idxg], rows, sem).wait()
        pltpu.sync_copy(rows, out.at[(base // _G) + g])
        return ()

    lax.fori_loop(0, _NG, step, ())


@jax.jit
def kernel(x, tables):
    tab = tables.reshape(N_FIELDS * VOCAB, EMB)
    xf = x.reshape(BATCH * N_FIELDS)
    mesh = plsc.VectorSubcoreMesh(core_axis_name="c", subcore_axis_name="s")
    run = functools.partial(
        pl.kernel,
        mesh=mesh,
        compiler_params=pltpu.CompilerParams(use_tc_tiling_on_sc=False),
        out_type=jax.ShapeDtypeStruct(
            (BATCH * N_FIELDS // _G, _G, EMB), jnp.float32
        ),
        scratch_types=[
            pltpu.VMEM((_ROWS_W,), jnp.int32),
            pltpu.VMEM((_G,), jnp.int32),
            pltpu.VMEM((_G, EMB), jnp.float32),
            pltpu.SemaphoreType.DMA,
        ],
    )(_body)
    out = run(tab, xf)
    return out.reshape(BATCH, N_FIELDS * EMB)


# speedup vs baseline: 1.0565x; 1.0381x over previous
import functools

import jax
import jax.numpy as jnp
from jax import lax
from jax.experimental import pallas as pl
from jax.experimental.pallas import tpu as pltpu
from jax.experimental.pallas import tpu_sc as plsc

N_FIELDS = 26
VOCAB = 100000
EMB = 32
BATCH = 16384

_NW = 32
_ROWS_W = BATCH * N_FIELDS // _NW
_G = 128
_NG = _ROWS_W // _G


def _body(tab, xf, out, xv, idx2, rows2, sem0, sem1):
    wid = lax.axis_index("s") * 2 + lax.axis_index("c")
    base = wid * _ROWS_W
    pltpu.sync_copy(xf.at[pl.ds(base, _ROWS_W)], xv)

    lane = lax.iota(jnp.int32, 16)
    sems = (sem0, sem1)

    def fire(g, buf):
        for s in range(_G // 16):
            j0 = g * _G + s * 16
            pos = j0 + lane
            off = (pos % N_FIELDS) * VOCAB
            idx2[buf, pl.ds(s * 16, 16)] = xv[pl.ds(j0, 16)] + off
        return pltpu.async_copy(
            tab.at[idx2.at[buf]], rows2.at[buf], sems[buf]
        )

    def finish(g, buf):
        pltpu.make_async_copy(
            tab.at[idx2.at[buf]], rows2.at[buf], sems[buf]
        ).wait()
        pltpu.sync_copy(rows2.at[buf], out.at[(base // _G) + g])

    fire(0, 0)

    def pair(g2, _):
        g = g2 * 2
        fire(g + 1, 1)
        finish(g, 0)

        @pl.when(g2 + 1 < _NG // 2)
        def _():
            fire(g + 2, 0)

        finish(g + 1, 1)
        return ()

    lax.fori_loop(0, _NG // 2, pair, ())


@jax.jit
def kernel(x, tables):
    tab = tables.reshape(N_FIELDS * VOCAB, EMB)
    xf = x.reshape(BATCH * N_FIELDS)
    mesh = plsc.VectorSubcoreMesh(core_axis_name="c", subcore_axis_name="s")
    run = functools.partial(
        pl.kernel,
        mesh=mesh,
        compiler_params=pltpu.CompilerParams(use_tc_tiling_on_sc=False),
        out_type=jax.ShapeDtypeStruct(
            (BATCH * N_FIELDS // _G, _G, EMB), jnp.float32
        ),
        scratch_types=[
            pltpu.VMEM((_ROWS_W,), jnp.int32),
            pltpu.VMEM((2, _G), jnp.int32),
            pltpu.VMEM((2, _G, EMB), jnp.float32),
            pltpu.SemaphoreType.DMA,
            pltpu.SemaphoreType.DMA,
        ],
    )(_body)
    out = run(tab, xf)
    return out.reshape(BATCH, N_FIELDS * EMB)
